# baseline (speedup 1.0000x reference)
import jax
import jax.numpy as jnp
from jax import lax
from jax.experimental import pallas as pl
from jax.experimental.pallas import tpu as pltpu

_N_WORDS = 100000
_DIM = 16
_CNT_ROWS = 782
_CPAD = _CNT_ROWS * 128 - _N_WORDS
_NCOVCH = 8


def _body(ix_ref, vecb_in, vhi_in, vlo_in, sum_h, cnt_h, cov_h,
          sum_o, cnt_o, cov_o, sum_v, cnt_v, cov_v, semb, semu):
    ix = ix_ref[0]
    t = ix // 128
    c = ix % 128
    cr = (ix // 1024) * 8

    bulk = [
        pltpu.make_async_copy(
            cov_h.at[pl.ds(32 * j, 32)], cov_o.at[pl.ds(32 * j, 32)], semb)
        for j in range(_NCOVCH)
    ]
    bulk.append(pltpu.make_async_copy(sum_h, sum_o, semb))
    bulk.append(pltpu.make_async_copy(cnt_h, cnt_o, semb))
    for b in bulk:
        b.start()

    fs = pltpu.make_async_copy(sum_h.at[:, pl.ds(t * 128, 128)], sum_v, semu)
    fc = pltpu.make_async_copy(cnt_h.at[pl.ds(cr, 8)], cnt_v, semu)
    fv = pltpu.make_async_copy(cov_h.at[:, pl.ds(t * 128, 128)], cov_v, semu)
    fs.start()
    fc.start()
    fv.start()
    fs.wait()
    fc.wait()
    fv.wait()

    lane16 = lax.broadcasted_iota(jnp.int32, (_DIM, 128), 1)
    sum_v[...] = sum_v[...] + jnp.where(lane16 == c, vecb_in[...], 0.0)

    r2 = (ix // 128) % 8
    row8 = lax.broadcasted_iota(jnp.int32, (8, 128), 0)
    lane8 = lax.broadcasted_iota(jnp.int32, (8, 128), 1)
    hit_c = jnp.logical_and(row8 == r2, lane8 == c)
    cnt_v[...] = cnt_v[...] + hit_c.astype(jnp.int32)

    lane256 = lax.broadcasted_iota(jnp.int32, (16 * _DIM, 128), 1)
    outer = vhi_in[...] * vlo_in[...]
    cov_v[...] = cov_v[...] + jnp.where(lane256 == c, outer, 0.0)

    for b in bulk:
        b.wait()

    ws = pltpu.make_async_copy(sum_v, sum_o.at[:, pl.ds(t * 128, 128)], semu)
    wc = pltpu.make_async_copy(cnt_v, cnt_o.at[pl.ds(cr, 8)], semu)
    wv = pltpu.make_async_copy(cov_v, cov_o.at[:, pl.ds(t * 128, 128)], semu)
    ws.start()
    wc.start()
    wv.start()
    ws.wait()
    wc.wait()
    wv.wait()


def kernel(ix, vec, sum_buf, counts, cov_buf):
    ix_arr = jnp.reshape(jnp.asarray(ix, jnp.int32), (1,))
    sum_t = jnp.transpose(sum_buf, (1, 0))
    cov_t = jnp.transpose(cov_buf, (1, 2, 0)).reshape(16 * _DIM, _N_WORDS)
    cpad = jnp.concatenate(
        [counts, jnp.zeros((_CPAD,), jnp.int32)]).reshape(_CNT_ROWS, 128)
    vecb = jnp.broadcast_to(vec.reshape(_DIM, 1), (_DIM, 128))
    vhi = jnp.broadcast_to(
        vec.reshape(_DIM, 1, 1), (_DIM, _DIM, 128)).reshape(16 * _DIM, 128)
    vlo = jnp.broadcast_to(
        vec.reshape(1, _DIM, 1), (_DIM, _DIM, 128)).reshape(16 * _DIM, 128)
    hbm = pl.BlockSpec(memory_space=pltpu.MemorySpace.HBM)
    grid_spec = pltpu.PrefetchScalarGridSpec(
        num_scalar_prefetch=1,
        grid=(1,),
        in_specs=[
            pl.BlockSpec((_DIM, 128), lambda i, s: (0, 0)),
            pl.BlockSpec((16 * _DIM, 128), lambda i, s: (0, 0)),
            pl.BlockSpec((16 * _DIM, 128), lambda i, s: (0, 0)),
            hbm,
            hbm,
            hbm,
        ],
        out_specs=[hbm, hbm, hbm],
        scratch_shapes=[
            pltpu.VMEM((_DIM, 128), jnp.float32),
            pltpu.VMEM((8, 128), jnp.int32),
            pltpu.VMEM((16 * _DIM, 128), jnp.float32),
            pltpu.SemaphoreType.DMA,
            pltpu.SemaphoreType.DMA,
        ],
    )
    out = pl.pallas_call(
        _body,
        grid_spec=grid_spec,
        out_shape=[
            jax.ShapeDtypeStruct((_DIM, _N_WORDS), jnp.float32),
            jax.ShapeDtypeStruct((_CNT_ROWS, 128), jnp.int32),
            jax.ShapeDtypeStruct((16 * _DIM, _N_WORDS), jnp.float32),
        ],
    )(ix_arr, vecb, vhi, vlo, sum_t, cpad, cov_t)
    return (jnp.transpose(out[0], (1, 0)),
            out[1].reshape(-1)[:_N_WORDS],
            jnp.transpose(out[2].reshape(_DIM, _DIM, _N_WORDS), (2, 0, 1)))


# speedup vs baseline: 45.0114x; 45.0114x over previous
import jax
import jax.numpy as jnp
from jax import lax
from jax.experimental import pallas as pl
from jax.experimental.pallas import tpu as pltpu

_N_WORDS = 100000
_DIM = 16
_CNT_ROWS = 782
_CPAD = _CNT_ROWS * 128 - _N_WORDS


def _body(ix_ref, vecb_in, vhi_in, vlo_in, sum_in, cnt_in, cov_in,
          sum_out, cnt_out, cov_out):
    ix = ix_ref[0]
    c = ix % 128

    lane16 = lax.broadcasted_iota(jnp.int32, (_DIM, 128), 1)
    sum_out[...] = sum_in[...] + jnp.where(lane16 == c, vecb_in[...], 0.0)

    r2 = (ix // 128) % 8
    row8 = lax.broadcasted_iota(jnp.int32, (8, 128), 0)
    lane8 = lax.broadcasted_iota(jnp.int32, (8, 128), 1)
    hit_c = jnp.logical_and(row8 == r2, lane8 == c)
    cnt_out[...] = cnt_in[...] + hit_c.astype(jnp.int32)

    lane256 = lax.broadcasted_iota(jnp.int32, (16 * _DIM, 128), 1)
    outer = vhi_in[...] * vlo_in[...]
    cov_out[...] = cov_in[...] + jnp.where(lane256 == c, outer, 0.0)


def kernel(ix, vec, sum_buf, counts, cov_buf):
    ix_arr = jnp.reshape(jnp.asarray(ix, jnp.int32), (1,))
    sum_t = jnp.transpose(sum_buf, (1, 0))
    cov_t = jnp.transpose(cov_buf, (1, 2, 0)).reshape(16 * _DIM, _N_WORDS)
    cpad = jnp.concatenate(
        [counts, jnp.zeros((_CPAD,), jnp.int32)]).reshape(_CNT_ROWS, 128)
    vecb = jnp.broadcast_to(vec.reshape(_DIM, 1), (_DIM, 128))
    vhi = jnp.broadcast_to(
        vec.reshape(_DIM, 1, 1), (_DIM, _DIM, 128)).reshape(16 * _DIM, 128)
    vlo = jnp.broadcast_to(
        vec.reshape(1, _DIM, 1), (_DIM, _DIM, 128)).reshape(16 * _DIM, 128)
    grid_spec = pltpu.PrefetchScalarGridSpec(
        num_scalar_prefetch=1,
        grid=(1,),
        in_specs=[
            pl.BlockSpec((_DIM, 128), lambda i, s: (0, 0)),
            pl.BlockSpec((16 * _DIM, 128), lambda i, s: (0, 0)),
            pl.BlockSpec((16 * _DIM, 128), lambda i, s: (0, 0)),
            pl.BlockSpec((_DIM, 128), lambda i, s: (0, s[0] // 128)),
            pl.BlockSpec((8, 128), lambda i, s: (s[0] // 1024, 0)),
            pl.BlockSpec((16 * _DIM, 128), lambda i, s: (0, s[0] // 128)),
        ],
        out_specs=[
            pl.BlockSpec((_DIM, 128), lambda i, s: (0, s[0] // 128)),
            pl.BlockSpec((8, 128), lambda i, s: (s[0] // 1024, 0)),
            pl.BlockSpec((16 * _DIM, 128), lambda i, s: (0, s[0] // 128)),
        ],
    )
    out = pl.pallas_call(
        _body,
        grid_spec=grid_spec,
        out_shape=[
            jax.ShapeDtypeStruct((_DIM, _N_WORDS), jnp.float32),
            jax.ShapeDtypeStruct((_CNT_ROWS, 128), jnp.int32),
            jax.ShapeDtypeStruct((16 * _DIM, _N_WORDS), jnp.float32),
        ],
        input_output_aliases={4: 0, 5: 1, 6: 2},
    )(ix_arr, vecb, vhi, vlo, sum_t, cpad, cov_t)
    return (jnp.transpose(out[0], (1, 0)),
            out[1].reshape(-1)[:_N_WORDS],
            jnp.transpose(out[2].reshape(_DIM, _DIM, _N_WORDS), (2, 0, 1)))
